# baseline (speedup 1.0000x reference)
import functools

import jax
import jax.numpy as jnp
from jax import lax
from jax.experimental import pallas as pl
from jax.experimental.pallas import tpu as pltpu
from jax.experimental.pallas import tpu_sc as plsc

N_NODES = 10000
N_ACC = 10240
NUM_CORES = 2
NUM_SUBCORES = 16
K_EDGES = 128
ROWS_PER_TILE = N_ACC // NUM_SUBCORES
CPB = 4

_HIGH = jax.lax.Precision.HIGHEST


def _dot(a, b):
    return jax.lax.dot_general(a, b, (((1,), (0,)), ((), ())),
                               precision=_HIGH,
                               preferred_element_type=jnp.float32)


def _sc_segment_sum(table, gidx, sidx, zeros, d, n_chunks, with_deg,
                    zeros16=None):
    mesh = plsc.VectorSubcoreMesh(core_axis_name="c", subcore_axis_name="s")
    n_blocks = n_chunks // CPB
    assert n_chunks % (2 * CPB) == 0

    out_type = [jax.ShapeDtypeStruct((NUM_CORES, N_ACC, d), jnp.float32)]
    scratch = (
        [pltpu.VMEM((CPB, K_EDGES), jnp.int32) for _ in range(4)]
        + [pltpu.VMEM((K_EDGES, d), jnp.float32) for _ in range(CPB)]
        + [pltpu.VMEM_SHARED((N_ACC, d), jnp.float32),
           pltpu.VMEM_SHARED((N_ACC, d), jnp.float32)]
        + [pltpu.SemaphoreType.DMA for _ in range(CPB + 2)]
    )
    if with_deg:
        out_type.append(
            jax.ShapeDtypeStruct((NUM_CORES, N_ACC, 16), jnp.float32))
        scratch += [pltpu.VMEM((K_EDGES, 16), jnp.float32),
                    pltpu.VMEM_SHARED((N_ACC, 16), jnp.float32)]

    @functools.partial(
        pl.kernel,
        out_type=out_type,
        mesh=mesh,
        scratch_types=scratch,
        compiler_params=pltpu.CompilerParams(use_tc_tiling_on_sc=False),
    )
    def seg_kernel(*args):
        if with_deg:
            (table_hbm, gidx_hbm, sidx_hbm, zeros_hbm, z16_hbm,
             out_hbm, deg_hbm, *rest) = args
            ones_v, deg_sh = rest[-2:]
            rest = rest[:-2]
        else:
            table_hbm, gidx_hbm, sidx_hbm, zeros_hbm, out_hbm, *rest = args
        gbuf = rest[0:2]
        dbuf = rest[2:4]
        rows = rest[4:4 + CPB]
        acc_sh = rest[4 + CPB]
        tbl_sh = rest[5 + CPB]
        gsem = rest[6 + CPB:6 + 2 * CPB]
        isem = rest[6 + 2 * CPB:6 + 2 * CPB + 2]
        c = lax.axis_index("c")
        s = lax.axis_index("s")
        wid = s
        tbl = table_hbm.at[c]
        rslc = pl.ds(s * ROWS_PER_TILE, ROWS_PER_TILE)
        pltpu.sync_copy(zeros_hbm.at[rslc], acc_sh.at[rslc])
        pltpu.sync_copy(tbl.at[rslc], tbl_sh.at[rslc])
        if with_deg:
            pltpu.sync_copy(z16_hbm.at[rslc], deg_sh.at[rslc])

            @pl.loop(0, K_EDGES)
            def _(i):
                ones_v[i] = jnp.full((16,), 1.0, jnp.float32)

        pltpu.sync_copy(gidx_hbm.at[wid, pl.ds(0, CPB)], gbuf[0])
        pltpu.sync_copy(sidx_hbm.at[wid, pl.ds(0, CPB)], dbuf[0])
        pltpu.make_async_copy(gidx_hbm.at[wid, pl.ds(CPB, CPB)], gbuf[1],
                              isem[1]).start()
        pltpu.make_async_copy(sidx_hbm.at[wid, pl.ds(CPB, CPB)], dbuf[1],
                              isem[1]).start()
        plsc.subcore_barrier()

        for j in range(CPB):
            pltpu.make_async_copy(tbl_sh.at[gbuf[0].at[j]], rows[j],
                                  gsem[j]).start()

        def half_step(blk_off, p):
            q = 1 - p
            pltpu.make_async_copy(gidx_hbm.at[wid, pl.ds(0, CPB)], gbuf[q],
                                  isem[q]).wait()
            pltpu.make_async_copy(sidx_hbm.at[wid, pl.ds(0, CPB)], dbuf[q],
                                  isem[q]).wait()
            for j in range(CPB):
                pltpu.make_async_copy(tbl_sh.at[gbuf[p].at[j]], rows[j],
                                      gsem[j]).wait()
                pltpu.sync_copy(rows[j], acc_sh.at[dbuf[p].at[j]], add=True)
                if with_deg:
                    pltpu.sync_copy(ones_v, deg_sh.at[dbuf[p].at[j]],
                                    add=True)
                pltpu.make_async_copy(tbl_sh.at[gbuf[q].at[j]], rows[j],
                                      gsem[j]).start()
            nxt = pl.ds((blk_off + 2) * CPB, CPB)
            pltpu.make_async_copy(gidx_hbm.at[wid, nxt], gbuf[p],
                                  isem[p]).start()
            pltpu.make_async_copy(sidx_hbm.at[wid, nxt], dbuf[p],
                                  isem[p]).start()

        @pl.loop(0, n_blocks // 2)
        def _(t):
            half_step(2 * t, 0)
            half_step(2 * t + 1, 1)

        for j in range(CPB):
            pltpu.make_async_copy(tbl_sh.at[gbuf[0].at[j]], rows[j],
                                  gsem[j]).wait()
        pltpu.make_async_copy(gidx_hbm.at[wid, pl.ds(0, CPB)], gbuf[1],
                              isem[1]).wait()
        pltpu.make_async_copy(sidx_hbm.at[wid, pl.ds(0, CPB)], dbuf[1],
                              isem[1]).wait()

        plsc.subcore_barrier()
        pltpu.sync_copy(acc_sh.at[rslc], out_hbm.at[c, rslc])
        if with_deg:
            pltpu.sync_copy(deg_sh.at[rslc], deg_hbm.at[c, rslc])

    if with_deg:
        return seg_kernel(table, gidx, sidx, zeros, zeros16)
    return seg_kernel(table, gidx, sidx, zeros)


ROW_BLK = 1000


def _tc_project(x, Ws, Wn):
    n, d_in = x.shape
    d_s = Ws.shape[1]
    d_n = Wn.shape[1]

    def body(x_ref, ws_ref, wn_ref, hs_ref, hw_ref):
        xb = x_ref[...]
        hs_ref[...] = _dot(xb, ws_ref[...])
        hw_ref[...] = _dot(xb, wn_ref[...])

    return pl.pallas_call(
        body,
        grid=(n // ROW_BLK,),
        in_specs=[
            pl.BlockSpec((ROW_BLK, d_in), lambda i: (i, 0)),
            pl.BlockSpec((d_in, d_s), lambda i: (0, 0)),
            pl.BlockSpec((d_in, d_n), lambda i: (0, 0)),
        ],
        out_specs=[
            pl.BlockSpec((ROW_BLK, d_s), lambda i: (i, 0)),
            pl.BlockSpec((ROW_BLK, d_n), lambda i: (i, 0)),
        ],
        out_shape=[
            jax.ShapeDtypeStruct((n, d_s), jnp.float32),
            jax.ShapeDtypeStruct((n, d_n), jnp.float32),
        ],
    )(x, Ws, Wn)


def _tc_combine_project(hs, acc, deg, b, Ws_next, Wn_next):
    n, d = hs.shape
    d_s = Ws_next.shape[1]
    d_n = Wn_next.shape[1]
    d_acc = acc.shape[2]

    def body(hs_ref, acc_ref, deg_ref, b_ref, ws_ref, wn_ref, hs2_ref, hw2_ref):
        inv = 1.0 / jnp.maximum(deg_ref[0, :, :1], 1.0)
        neigh = jnp.concatenate([acc_ref[0], acc_ref[1]], axis=-1) * inv
        h = jnp.maximum(hs_ref[...] + neigh + b_ref[...], 0.0)
        hs2_ref[...] = _dot(h, ws_ref[...])
        hw2_ref[...] = _dot(h, wn_ref[...])

    return pl.pallas_call(
        body,
        grid=(n // ROW_BLK,),
        in_specs=[
            pl.BlockSpec((ROW_BLK, d), lambda i: (i, 0)),
            pl.BlockSpec((2, ROW_BLK, d_acc), lambda i: (0, i, 0)),
            pl.BlockSpec((1, ROW_BLK, 16), lambda i: (0, i, 0)),
            pl.BlockSpec((1, d), lambda i: (0, 0)),
            pl.BlockSpec((d, d_s), lambda i: (0, 0)),
            pl.BlockSpec((d, d_n), lambda i: (0, 0)),
        ],
        out_specs=[
            pl.BlockSpec((ROW_BLK, d_s), lambda i: (i, 0)),
            pl.BlockSpec((ROW_BLK, d_n), lambda i: (i, 0)),
        ],
        out_shape=[
            jax.ShapeDtypeStruct((n, d_s), jnp.float32),
            jax.ShapeDtypeStruct((n, d_n), jnp.float32),
        ],
    )(hs, acc, deg, b, Ws_next, Wn_next)


def _tc_final(hs3, acc3, deg, b3, pW1, pb1, pW2, pb2, pW3, pb3):
    n, d = hs3.shape
    a_dim = pW3.shape[1]

    def body(hs_ref, acc_ref, deg_ref, b_ref, w1_ref, b1_ref, w2_ref, b2_ref,
             w3_ref, b3_ref, out_ref):
        inv = 1.0 / jnp.maximum(deg_ref[0, :n, :1], 1.0)
        neigh = jnp.concatenate([acc_ref[0, :n], acc_ref[1, :n]],
                                axis=-1) * inv
        h = jnp.maximum(hs_ref[...] + neigh + b_ref[...], 0.0)
        g = jnp.sum(h, axis=0, keepdims=True) * (1.0 / n)
        l1 = jnp.maximum(_dot(g, w1_ref[...]) + b1_ref[...], 0.0)
        l2 = jnp.maximum(_dot(l1, w2_ref[...]) + b2_ref[...], 0.0)
        out_ref[...] = _dot(l2, w3_ref[...]) + b3_ref[...]

    return pl.pallas_call(
        body,
        out_shape=jax.ShapeDtypeStruct((1, a_dim), jnp.float32),
    )(hs3, acc3, deg, b3, pW1, pb1, pW2, pb2, pW3, pb3)


def kernel(x, edge_index, Ws1, Wn1, b1, Ws2, Wn2, b2, Ws3, Wn3, b3,
           pW1, pb1, pW2, pb2, pW3, pb3):
    n = x.shape[0]
    e = edge_index.shape[1]
    src = edge_index[0]
    dst = edge_index[1]

    n_chunks = -(-e // (NUM_SUBCORES * K_EDGES))
    n_chunks = -(-n_chunks // (2 * CPB)) * (2 * CPB)
    e_pad = NUM_SUBCORES * K_EDGES * n_chunks
    pad = e_pad - e
    pad_rows = n + (jnp.arange(pad, dtype=jnp.int32) % (N_ACC - n))
    idx_c = (NUM_SUBCORES, n_chunks, K_EDGES)
    ipad = ((0, 0), (0, 2 * CPB), (0, 0))
    src_c = jnp.pad(
        jnp.concatenate([src, jnp.zeros((pad,), jnp.int32)]).reshape(idx_c),
        ipad)
    dst_c = jnp.pad(
        jnp.concatenate([dst, pad_rows]).reshape(idx_c), ipad)

    zeros64 = jnp.zeros((N_ACC, 64), jnp.float32)
    zeros32 = jnp.zeros((N_ACC, 32), jnp.float32)
    zeros16 = jnp.zeros((N_ACC, 16), jnp.float32)
    tpad64 = jnp.zeros((N_ACC - n, 64), jnp.float32)
    tpad32 = jnp.zeros((N_ACC - n, 32), jnp.float32)

    b1r = b1.reshape(1, -1)
    b2r = b2.reshape(1, -1)
    b3r = b3.reshape(1, -1)

    hs1, hw1 = _tc_project(x, Ws1, Wn1)
    hw1h = jnp.stack([jnp.concatenate([hw1[:, :64], tpad64]),
                      jnp.concatenate([hw1[:, 64:], tpad64])])
    acc1, deg = _sc_segment_sum(hw1h, src_c, dst_c, zeros64, 64, n_chunks,
                                with_deg=True, zeros16=zeros16)
    hs2, hw2 = _tc_combine_project(hs1, acc1, deg, b1r, Ws2, Wn2)
    hw2h = jnp.stack([jnp.concatenate([hw2[:, :64], tpad64]),
                      jnp.concatenate([hw2[:, 64:], tpad64])])
    acc2 = _sc_segment_sum(hw2h, src_c, dst_c, zeros64, 64, n_chunks,
                           with_deg=False)[0]
    hs3, hw3 = _tc_combine_project(hs2, acc2, deg, b2r, Ws3, Wn3)
    hw3h = jnp.stack([jnp.concatenate([hw3[:, :32], tpad32]),
                      jnp.concatenate([hw3[:, 32:], tpad32])])
    acc3 = _sc_segment_sum(hw3h, src_c, dst_c, zeros32, 32, n_chunks,
                           with_deg=False)[0]
    logits = _tc_final(hs3, acc3, deg, b3r,
                       pW1, pb1.reshape(1, -1), pW2, pb2.reshape(1, -1),
                       pW3, pb3.reshape(1, -1))
    return logits[0]


# speedup vs baseline: 1.0759x; 1.0759x over previous
import functools

import jax
import jax.numpy as jnp
from jax import lax
from jax.experimental import pallas as pl
from jax.experimental.pallas import tpu as pltpu
from jax.experimental.pallas import tpu_sc as plsc

N_NODES = 10000
N_ACC = 10240
NUM_CORES = 2
NUM_SUBCORES = 16
K_EDGES = 128
ROWS_PER_TILE = N_ACC // NUM_SUBCORES
CPB = 4

_HIGH = jax.lax.Precision.HIGHEST


def _dot(a, b):
    return jax.lax.dot_general(a, b, (((1,), (0,)), ((), ())),
                               precision=_HIGH,
                               preferred_element_type=jnp.float32)


def _sc_segment_sum(table, gidx, sidx, zeros, d, n_chunks):
    mesh = plsc.VectorSubcoreMesh(core_axis_name="c", subcore_axis_name="s")
    n_blocks = n_chunks // CPB
    assert n_chunks % (2 * CPB) == 0

    @functools.partial(
        pl.kernel,
        out_type=jax.ShapeDtypeStruct((NUM_CORES, N_ACC, d), jnp.float32),
        mesh=mesh,
        scratch_types=(
            [pltpu.VMEM((CPB, K_EDGES), jnp.int32) for _ in range(4)]
            + [pltpu.VMEM((K_EDGES, d), jnp.float32) for _ in range(CPB)]
            + [pltpu.VMEM_SHARED((N_ACC, d), jnp.float32),
               pltpu.VMEM_SHARED((N_ACC, d), jnp.float32)]
            + [pltpu.SemaphoreType.DMA for _ in range(CPB + 2)]
        ),
        compiler_params=pltpu.CompilerParams(use_tc_tiling_on_sc=False),
    )
    def seg_kernel(table_hbm, gidx_hbm, sidx_hbm, zeros_hbm, out_hbm, *rest):
        gbuf = rest[0:2]
        dbuf = rest[2:4]
        rows = rest[4:4 + CPB]
        acc_sh = rest[4 + CPB]
        tbl_sh = rest[5 + CPB]
        gsem = rest[6 + CPB:6 + 2 * CPB]
        isem = rest[6 + 2 * CPB:]
        c = lax.axis_index("c")
        s = lax.axis_index("s")
        tbl = table_hbm.at[c]
        rslc = pl.ds(s * ROWS_PER_TILE, ROWS_PER_TILE)
        pltpu.sync_copy(zeros_hbm.at[rslc], acc_sh.at[rslc])
        pltpu.sync_copy(tbl.at[rslc], tbl_sh.at[rslc])
        pltpu.sync_copy(gidx_hbm.at[s, pl.ds(0, CPB)], gbuf[0])
        pltpu.sync_copy(sidx_hbm.at[s, pl.ds(0, CPB)], dbuf[0])
        pltpu.make_async_copy(gidx_hbm.at[s, pl.ds(CPB, CPB)], gbuf[1],
                              isem[1]).start()
        pltpu.make_async_copy(sidx_hbm.at[s, pl.ds(CPB, CPB)], dbuf[1],
                              isem[1]).start()
        plsc.subcore_barrier()

        for j in range(CPB):
            pltpu.make_async_copy(tbl_sh.at[gbuf[0].at[j]], rows[j],
                                  gsem[j]).start()

        def half_step(blk_off, p):
            q = 1 - p
            pltpu.make_async_copy(gidx_hbm.at[s, pl.ds(0, CPB)], gbuf[q],
                                  isem[q]).wait()
            pltpu.make_async_copy(sidx_hbm.at[s, pl.ds(0, CPB)], dbuf[q],
                                  isem[q]).wait()
            for j in range(CPB):
                pltpu.make_async_copy(tbl_sh.at[gbuf[p].at[j]], rows[j],
                                      gsem[j]).wait()
                pltpu.sync_copy(rows[j], acc_sh.at[dbuf[p].at[j]], add=True)
                pltpu.make_async_copy(tbl_sh.at[gbuf[q].at[j]], rows[j],
                                      gsem[j]).start()
            nxt = pl.ds(lax.rem(blk_off + 2, n_blocks) * CPB, CPB)
            pltpu.make_async_copy(gidx_hbm.at[s, nxt], gbuf[p],
                                  isem[p]).start()
            pltpu.make_async_copy(sidx_hbm.at[s, nxt], dbuf[p],
                                  isem[p]).start()

        @pl.loop(0, n_blocks // 2)
        def _(t):
            half_step(2 * t, 0)
            half_step(2 * t + 1, 1)

        for j in range(CPB):
            pltpu.make_async_copy(tbl_sh.at[gbuf[0].at[j]], rows[j],
                                  gsem[j]).wait()
        pltpu.make_async_copy(gidx_hbm.at[s, pl.ds(0, CPB)], gbuf[1],
                              isem[1]).wait()
        pltpu.make_async_copy(sidx_hbm.at[s, pl.ds(0, CPB)], dbuf[1],
                              isem[1]).wait()

        plsc.subcore_barrier()
        pltpu.sync_copy(acc_sh.at[rslc], out_hbm.at[c, rslc])

    return seg_kernel(table, gidx, sidx, zeros)


def _sc_degree(sidx, zeros16, n_chunks):
    mesh = plsc.VectorSubcoreMesh(core_axis_name="c", subcore_axis_name="s")
    n_blocks = n_chunks // CPB

    @functools.partial(
        pl.kernel,
        out_type=jax.ShapeDtypeStruct((NUM_CORES, N_ACC, 16), jnp.float32),
        mesh=mesh,
        scratch_types=(
            [pltpu.VMEM((CPB, K_EDGES), jnp.int32) for _ in range(2)]
            + [pltpu.VMEM((K_EDGES, 16), jnp.float32),
               pltpu.VMEM_SHARED((N_ACC, 16), jnp.float32)]
            + [pltpu.SemaphoreType.DMA for _ in range(2)]
        ),
        compiler_params=pltpu.CompilerParams(use_tc_tiling_on_sc=False),
    )
    def deg_kernel(sidx_hbm, zeros_hbm, out_hbm, dbuf0, dbuf1, ones_v,
                   deg_sh, isem0, isem1):
        dbuf = (dbuf0, dbuf1)
        isem = (isem0, isem1)
        c = lax.axis_index("c")
        s = lax.axis_index("s")
        rslc = pl.ds(s * ROWS_PER_TILE, ROWS_PER_TILE)
        pltpu.sync_copy(zeros_hbm.at[rslc], deg_sh.at[rslc])

        @pl.loop(0, K_EDGES)
        def _(i):
            ones_v[i] = jnp.full((16,), 1.0, jnp.float32)

        pltpu.sync_copy(sidx_hbm.at[s, pl.ds(0, CPB)], dbuf[0])
        pltpu.make_async_copy(sidx_hbm.at[s, pl.ds(CPB, CPB)], dbuf[1],
                              isem[1]).start()
        plsc.subcore_barrier()

        def half_step(blk_off, p):
            q = 1 - p
            pltpu.make_async_copy(sidx_hbm.at[s, pl.ds(0, CPB)], dbuf[q],
                                  isem[q]).wait()
            for j in range(CPB):
                pltpu.sync_copy(ones_v, deg_sh.at[dbuf[p].at[j]], add=True)
            nxt = pl.ds(lax.rem(blk_off + 2, n_blocks) * CPB, CPB)
            pltpu.make_async_copy(sidx_hbm.at[s, nxt], dbuf[p],
                                  isem[p]).start()

        @pl.loop(0, n_blocks // 2)
        def _(t):
            half_step(2 * t, 0)
            half_step(2 * t + 1, 1)

        pltpu.make_async_copy(sidx_hbm.at[s, pl.ds(0, CPB)], dbuf[1],
                              isem[1]).wait()
        plsc.subcore_barrier()
        pltpu.sync_copy(deg_sh.at[rslc], out_hbm.at[c, rslc])

    return deg_kernel(sidx, zeros16)


ROW_BLK = 2000


def _tc_project(x, Ws, Wn):
    n, d_in = x.shape
    d_s = Ws.shape[1]
    d_n = Wn.shape[1]
    h2 = d_n // 2

    def body(x_ref, ws_ref, wn_ref, hs_ref, hw_ref):
        xb = x_ref[...]
        hs_ref[...] = _dot(xb, ws_ref[...])
        hw = _dot(xb, wn_ref[...])
        hw_ref[0] = hw[:, :h2]
        hw_ref[1] = hw[:, h2:]

    return pl.pallas_call(
        body,
        grid=(n // ROW_BLK,),
        in_specs=[
            pl.BlockSpec((ROW_BLK, d_in), lambda i: (i, 0)),
            pl.BlockSpec((d_in, d_s), lambda i: (0, 0)),
            pl.BlockSpec((d_in, d_n), lambda i: (0, 0)),
        ],
        out_specs=[
            pl.BlockSpec((ROW_BLK, d_s), lambda i: (i, 0)),
            pl.BlockSpec((2, ROW_BLK, h2), lambda i: (0, i, 0)),
        ],
        out_shape=[
            jax.ShapeDtypeStruct((n, d_s), jnp.float32),
            jax.ShapeDtypeStruct((2, N_ACC, h2), jnp.float32),
        ],
    )(x, Ws, Wn)


def _tc_combine_project(hs, acc, deg, b, Ws_next, Wn_next):
    n, d = hs.shape
    d_s = Ws_next.shape[1]
    d_n = Wn_next.shape[1]
    h2 = d_n // 2

    d_acc = acc.shape[2]

    def body(hs_ref, acc_ref, deg_ref, b_ref, ws_ref, wn_ref, hs2_ref,
             hw2_ref):
        inv = 1.0 / jnp.maximum(deg_ref[0, :, :1], 1.0)
        neigh = jnp.concatenate([acc_ref[0], acc_ref[1]], axis=-1) * inv
        h = jnp.maximum(hs_ref[...] + neigh + b_ref[...], 0.0)
        hs2_ref[...] = _dot(h, ws_ref[...])
        hw = _dot(h, wn_ref[...])
        hw2_ref[0] = hw[:, :h2]
        hw2_ref[1] = hw[:, h2:]

    return pl.pallas_call(
        body,
        grid=(n // ROW_BLK,),
        in_specs=[
            pl.BlockSpec((ROW_BLK, d), lambda i: (i, 0)),
            pl.BlockSpec((2, ROW_BLK, d_acc), lambda i: (0, i, 0)),
            pl.BlockSpec((1, ROW_BLK, 16), lambda i: (0, i, 0)),
            pl.BlockSpec((1, d), lambda i: (0, 0)),
            pl.BlockSpec((d, d_s), lambda i: (0, 0)),
            pl.BlockSpec((d, d_n), lambda i: (0, 0)),
        ],
        out_specs=[
            pl.BlockSpec((ROW_BLK, d_s), lambda i: (i, 0)),
            pl.BlockSpec((2, ROW_BLK, h2), lambda i: (0, i, 0)),
        ],
        out_shape=[
            jax.ShapeDtypeStruct((n, d_s), jnp.float32),
            jax.ShapeDtypeStruct((2, N_ACC, h2), jnp.float32),
        ],
    )(hs, acc, deg, b, Ws_next, Wn_next)


def _tc_final(hs3, acc3, deg, b3, pW1, pb1, pW2, pb2, pW3, pb3):
    n, d = hs3.shape
    a_dim = pW3.shape[1]

    def body(hs_ref, acc_ref, deg_ref, b_ref, w1_ref, b1_ref, w2_ref, b2_ref,
             w3_ref, b3_ref, out_ref):
        inv = 1.0 / jnp.maximum(deg_ref[0, :n, :1], 1.0)
        neigh = jnp.concatenate([acc_ref[0, :n], acc_ref[1, :n]],
                                axis=-1) * inv
        h = jnp.maximum(hs_ref[...] + neigh + b_ref[...], 0.0)
        g = jnp.sum(h, axis=0, keepdims=True) * (1.0 / n)
        l1 = jnp.maximum(_dot(g, w1_ref[...]) + b1_ref[...], 0.0)
        l2 = jnp.maximum(_dot(l1, w2_ref[...]) + b2_ref[...], 0.0)
        out_ref[...] = _dot(l2, w3_ref[...]) + b3_ref[...]

    return pl.pallas_call(
        body,
        out_shape=jax.ShapeDtypeStruct((1, a_dim), jnp.float32),
    )(hs3, acc3, deg, b3, pW1, pb1, pW2, pb2, pW3, pb3)


def kernel(x, edge_index, Ws1, Wn1, b1, Ws2, Wn2, b2, Ws3, Wn3, b3,
           pW1, pb1, pW2, pb2, pW3, pb3):
    n = x.shape[0]
    e = edge_index.shape[1]
    src = edge_index[0]
    dst = edge_index[1]

    n_chunks = -(-e // (NUM_SUBCORES * K_EDGES))
    n_chunks = -(-n_chunks // (2 * CPB)) * (2 * CPB)
    e_pad = NUM_SUBCORES * K_EDGES * n_chunks
    pad = e_pad - e
    pad_rows = n + (jnp.arange(pad, dtype=jnp.int32) % (N_ACC - n))
    idx_c = (NUM_SUBCORES, n_chunks, K_EDGES)
    src_c = jnp.concatenate([src, jnp.zeros((pad,), jnp.int32)]).reshape(idx_c)
    dst_c = jnp.concatenate([dst, pad_rows]).reshape(idx_c)

    zeros64 = jnp.zeros((N_ACC, 64), jnp.float32)
    zeros32 = jnp.zeros((N_ACC, 32), jnp.float32)
    zeros16 = jnp.zeros((N_ACC, 16), jnp.float32)

    b1r = b1.reshape(1, -1)
    b2r = b2.reshape(1, -1)
    b3r = b3.reshape(1, -1)

    deg = _sc_degree(dst_c, zeros16, n_chunks)

    hs1, hw1h = _tc_project(x, Ws1, Wn1)
    acc1 = _sc_segment_sum(hw1h, src_c, dst_c, zeros64, 64, n_chunks)
    hs2, hw2h = _tc_combine_project(hs1, acc1, deg, b1r, Ws2, Wn2)
    acc2 = _sc_segment_sum(hw2h, src_c, dst_c, zeros64, 64, n_chunks)
    hs3, hw3h = _tc_combine_project(hs2, acc2, deg, b2r, Ws3, Wn3)
    acc3 = _sc_segment_sum(hw3h, src_c, dst_c, zeros32, 32, n_chunks)
    logits = _tc_final(hs3, acc3, deg, b3r,
                       pW1, pb1.reshape(1, -1), pW2, pb2.reshape(1, -1),
                       pW3, pb3.reshape(1, -1))
    return logits[0]


# speedup vs baseline: 1.1893x; 1.1054x over previous
import functools

import jax
import jax.numpy as jnp
from jax import lax
from jax.experimental import pallas as pl
from jax.experimental.pallas import tpu as pltpu
from jax.experimental.pallas import tpu_sc as plsc

N_NODES = 10000
N_ACC = 10240
NUM_CORES = 2
NUM_SUBCORES = 16
K_EDGES = 128
ROWS_PER_TILE = N_ACC // NUM_SUBCORES
CPB = 4

_HIGH = jax.lax.Precision.HIGHEST


def _dot(a, b):
    return jax.lax.dot_general(a, b, (((1,), (0,)), ((), ())),
                               precision=_HIGH,
                               preferred_element_type=jnp.float32)


def _sc_segment_sum(table, gidx, sidx, zeros, d, n_chunks):
    mesh = plsc.VectorSubcoreMesh(core_axis_name="c", subcore_axis_name="s")
    n_blocks = n_chunks // CPB
    assert n_chunks % (2 * CPB) == 0

    @functools.partial(
        pl.kernel,
        out_type=jax.ShapeDtypeStruct((N_ACC, 2 * d), jnp.float32),
        mesh=mesh,
        scratch_types=(
            [pltpu.VMEM((CPB, K_EDGES), jnp.int32) for _ in range(4)]
            + [pltpu.VMEM((K_EDGES, d), jnp.float32) for _ in range(CPB)]
            + [pltpu.VMEM_SHARED((N_ACC, d), jnp.float32),
               pltpu.VMEM_SHARED((N_ACC, d), jnp.float32)]
            + [pltpu.SemaphoreType.DMA for _ in range(CPB + 2)]
        ),
        compiler_params=pltpu.CompilerParams(use_tc_tiling_on_sc=False),
    )
    def seg_kernel(table_hbm, gidx_hbm, sidx_hbm, zeros_hbm, out_hbm, *rest):
        gbuf = rest[0:2]
        dbuf = rest[2:4]
        rows = rest[4:4 + CPB]
        acc_sh = rest[4 + CPB]
        tbl_sh = rest[5 + CPB]
        gsem = rest[6 + CPB:6 + 2 * CPB]
        isem = rest[6 + 2 * CPB:]
        c = lax.axis_index("c")
        s = lax.axis_index("s")
        rslc = pl.ds(s * ROWS_PER_TILE, ROWS_PER_TILE)
        cslc = pl.ds(c * d, d)
        pltpu.sync_copy(zeros_hbm.at[rslc], acc_sh.at[rslc])
        pltpu.sync_copy(table_hbm.at[rslc, cslc], tbl_sh.at[rslc])
        pltpu.sync_copy(gidx_hbm.at[s, pl.ds(0, CPB)], gbuf[0])
        pltpu.sync_copy(sidx_hbm.at[s, pl.ds(0, CPB)], dbuf[0])
        pltpu.make_async_copy(gidx_hbm.at[s, pl.ds(CPB, CPB)], gbuf[1],
                              isem[1]).start()
        pltpu.make_async_copy(sidx_hbm.at[s, pl.ds(CPB, CPB)], dbuf[1],
                              isem[1]).start()
        plsc.subcore_barrier()

        for j in range(CPB):
            pltpu.make_async_copy(tbl_sh.at[gbuf[0].at[j]], rows[j],
                                  gsem[j]).start()

        def half_step(blk_off, p):
            q = 1 - p
            pltpu.make_async_copy(gidx_hbm.at[s, pl.ds(0, CPB)], gbuf[q],
                                  isem[q]).wait()
            pltpu.make_async_copy(sidx_hbm.at[s, pl.ds(0, CPB)], dbuf[q],
                                  isem[q]).wait()
            for j in range(CPB):
                pltpu.make_async_copy(tbl_sh.at[gbuf[p].at[j]], rows[j],
                                      gsem[j]).wait()
                pltpu.sync_copy(rows[j], acc_sh.at[dbuf[p].at[j]], add=True)
                pltpu.make_async_copy(tbl_sh.at[gbuf[q].at[j]], rows[j],
                                      gsem[j]).start()
            nxt = pl.ds(lax.rem(blk_off + 2, n_blocks) * CPB, CPB)
            pltpu.make_async_copy(gidx_hbm.at[s, nxt], gbuf[p],
                                  isem[p]).start()
            pltpu.make_async_copy(sidx_hbm.at[s, nxt], dbuf[p],
                                  isem[p]).start()

        @pl.loop(0, n_blocks // 2)
        def _(t):
            half_step(2 * t, 0)
            half_step(2 * t + 1, 1)

        for j in range(CPB):
            pltpu.make_async_copy(tbl_sh.at[gbuf[0].at[j]], rows[j],
                                  gsem[j]).wait()
        pltpu.make_async_copy(gidx_hbm.at[s, pl.ds(0, CPB)], gbuf[1],
                              isem[1]).wait()
        pltpu.make_async_copy(sidx_hbm.at[s, pl.ds(0, CPB)], dbuf[1],
                              isem[1]).wait()

        plsc.subcore_barrier()
        pltpu.sync_copy(acc_sh.at[rslc], out_hbm.at[rslc, cslc])

    return seg_kernel(table, gidx, sidx, zeros)


def _sc_degree(sidx, zeros16, n_chunks):
    mesh = plsc.VectorSubcoreMesh(core_axis_name="c", subcore_axis_name="s")
    n_blocks = n_chunks // CPB

    @functools.partial(
        pl.kernel,
        out_type=jax.ShapeDtypeStruct((NUM_CORES, N_ACC, 16), jnp.float32),
        mesh=mesh,
        scratch_types=(
            [pltpu.VMEM((CPB, K_EDGES), jnp.int32) for _ in range(2)]
            + [pltpu.VMEM((K_EDGES, 16), jnp.float32),
               pltpu.VMEM_SHARED((N_ACC, 16), jnp.float32)]
            + [pltpu.SemaphoreType.DMA for _ in range(2)]
        ),
        compiler_params=pltpu.CompilerParams(use_tc_tiling_on_sc=False),
    )
    def deg_kernel(sidx_hbm, zeros_hbm, out_hbm, dbuf0, dbuf1, ones_v,
                   deg_sh, isem0, isem1):
        dbuf = (dbuf0, dbuf1)
        isem = (isem0, isem1)
        c = lax.axis_index("c")
        s = lax.axis_index("s")
        rslc = pl.ds(s * ROWS_PER_TILE, ROWS_PER_TILE)
        pltpu.sync_copy(zeros_hbm.at[rslc], deg_sh.at[rslc])

        @pl.loop(0, K_EDGES)
        def _(i):
            ones_v[i] = jnp.full((16,), 1.0, jnp.float32)

        pltpu.sync_copy(sidx_hbm.at[s, pl.ds(0, CPB)], dbuf[0])
        pltpu.make_async_copy(sidx_hbm.at[s, pl.ds(CPB, CPB)], dbuf[1],
                              isem[1]).start()
        plsc.subcore_barrier()

        def half_step(blk_off, p):
            q = 1 - p
            pltpu.make_async_copy(sidx_hbm.at[s, pl.ds(0, CPB)], dbuf[q],
                                  isem[q]).wait()
            for j in range(CPB):
                pltpu.sync_copy(ones_v, deg_sh.at[dbuf[p].at[j]], add=True)
            nxt = pl.ds(lax.rem(blk_off + 2, n_blocks) * CPB, CPB)
            pltpu.make_async_copy(sidx_hbm.at[s, nxt], dbuf[p],
                                  isem[p]).start()

        @pl.loop(0, n_blocks // 2)
        def _(t):
            half_step(2 * t, 0)
            half_step(2 * t + 1, 1)

        pltpu.make_async_copy(sidx_hbm.at[s, pl.ds(0, CPB)], dbuf[1],
                              isem[1]).wait()
        plsc.subcore_barrier()
        pltpu.sync_copy(deg_sh.at[rslc], out_hbm.at[c, rslc])

    return deg_kernel(sidx, zeros16)


ROW_BLK = 2000


def _tc_project(x, Ws, Wn):
    n, d_in = x.shape
    d_s = Ws.shape[1]
    d_n = Wn.shape[1]
    h2 = d_n // 2

    def body(x_ref, ws_ref, wn_ref, hs_ref, hw_ref):
        xb = x_ref[...]
        hs_ref[...] = _dot(xb, ws_ref[...])
        hw_ref[...] = _dot(xb, wn_ref[...])

    return pl.pallas_call(
        body,
        grid=(n // ROW_BLK,),
        in_specs=[
            pl.BlockSpec((ROW_BLK, d_in), lambda i: (i, 0)),
            pl.BlockSpec((d_in, d_s), lambda i: (0, 0)),
            pl.BlockSpec((d_in, d_n), lambda i: (0, 0)),
        ],
        out_specs=[
            pl.BlockSpec((ROW_BLK, d_s), lambda i: (i, 0)),
            pl.BlockSpec((ROW_BLK, d_n), lambda i: (i, 0)),
        ],
        out_shape=[
            jax.ShapeDtypeStruct((n, d_s), jnp.float32),
            jax.ShapeDtypeStruct((N_ACC, d_n), jnp.float32),
        ],
    )(x, Ws, Wn)


def _tc_combine_project(hs, acc, deg, b, Ws_next, Wn_next):
    n, d = hs.shape
    d_s = Ws_next.shape[1]
    d_n = Wn_next.shape[1]
    h2 = d_n // 2

    def body(hs_ref, acc_ref, deg_ref, b_ref, ws_ref, wn_ref, hs2_ref,
             hw2_ref):
        inv = 1.0 / jnp.maximum(deg_ref[0, :, :1], 1.0)
        neigh = acc_ref[...] * inv
        h = jnp.maximum(hs_ref[...] + neigh + b_ref[...], 0.0)
        hs2_ref[...] = _dot(h, ws_ref[...])
        hw2_ref[...] = _dot(h, wn_ref[...])

    return pl.pallas_call(
        body,
        grid=(n // ROW_BLK,),
        in_specs=[
            pl.BlockSpec((ROW_BLK, d), lambda i: (i, 0)),
            pl.BlockSpec((ROW_BLK, d), lambda i: (i, 0)),
            pl.BlockSpec((1, ROW_BLK, 16), lambda i: (0, i, 0)),
            pl.BlockSpec((1, d), lambda i: (0, 0)),
            pl.BlockSpec((d, d_s), lambda i: (0, 0)),
            pl.BlockSpec((d, d_n), lambda i: (0, 0)),
        ],
        out_specs=[
            pl.BlockSpec((ROW_BLK, d_s), lambda i: (i, 0)),
            pl.BlockSpec((ROW_BLK, d_n), lambda i: (i, 0)),
        ],
        out_shape=[
            jax.ShapeDtypeStruct((n, d_s), jnp.float32),
            jax.ShapeDtypeStruct((N_ACC, d_n), jnp.float32),
        ],
    )(hs, acc, deg, b, Ws_next, Wn_next)


def _tc_final(hs3, acc3, deg, b3, pW1, pb1, pW2, pb2, pW3, pb3):
    n, d = hs3.shape
    a_dim = pW3.shape[1]

    def body(hs_ref, acc_ref, deg_ref, b_ref, w1_ref, b1_ref, w2_ref, b2_ref,
             w3_ref, b3_ref, out_ref):
        inv = 1.0 / jnp.maximum(deg_ref[0, :n, :1], 1.0)
        neigh = acc_ref[:n] * inv
        h = jnp.maximum(hs_ref[...] + neigh + b_ref[...], 0.0)
        g = jnp.sum(h, axis=0, keepdims=True) * (1.0 / n)
        l1 = jnp.maximum(_dot(g, w1_ref[...]) + b1_ref[...], 0.0)
        l2 = jnp.maximum(_dot(l1, w2_ref[...]) + b2_ref[...], 0.0)
        out_ref[...] = _dot(l2, w3_ref[...]) + b3_ref[...]

    return pl.pallas_call(
        body,
        out_shape=jax.ShapeDtypeStruct((1, a_dim), jnp.float32),
    )(hs3, acc3, deg, b3, pW1, pb1, pW2, pb2, pW3, pb3)


def kernel(x, edge_index, Ws1, Wn1, b1, Ws2, Wn2, b2, Ws3, Wn3, b3,
           pW1, pb1, pW2, pb2, pW3, pb3):
    n = x.shape[0]
    e = edge_index.shape[1]
    src = edge_index[0]
    dst = edge_index[1]

    n_chunks = -(-e // (NUM_SUBCORES * K_EDGES))
    n_chunks = -(-n_chunks // (2 * CPB)) * (2 * CPB)
    e_pad = NUM_SUBCORES * K_EDGES * n_chunks
    pad = e_pad - e
    pad_rows = n + (jnp.arange(pad, dtype=jnp.int32) % (N_ACC - n))
    idx_c = (NUM_SUBCORES, n_chunks, K_EDGES)
    src_c = jnp.concatenate([src, jnp.zeros((pad,), jnp.int32)]).reshape(idx_c)
    dst_c = jnp.concatenate([dst, pad_rows]).reshape(idx_c)

    zeros64 = jnp.zeros((N_ACC, 64), jnp.float32)
    zeros32 = jnp.zeros((N_ACC, 32), jnp.float32)
    zeros16 = jnp.zeros((N_ACC, 16), jnp.float32)

    b1r = b1.reshape(1, -1)
    b2r = b2.reshape(1, -1)
    b3r = b3.reshape(1, -1)

    deg = _sc_degree(dst_c, zeros16, n_chunks)

    hs1, hw1p = _tc_project(x, Ws1, Wn1)
    acc1 = _sc_segment_sum(hw1p, src_c, dst_c, zeros64, 64, n_chunks)
    hs2, hw2p = _tc_combine_project(hs1, acc1, deg, b1r, Ws2, Wn2)
    acc2 = _sc_segment_sum(hw2p, src_c, dst_c, zeros64, 64, n_chunks)
    hs3, hw3p = _tc_combine_project(hs2, acc2, deg, b2r, Ws3, Wn3)
    acc3 = _sc_segment_sum(hw3p, src_c, dst_c, zeros32, 32, n_chunks)
    logits = _tc_final(hs3, acc3, deg, b3r,
                       pW1, pb1.reshape(1, -1), pW2, pb2.reshape(1, -1),
                       pW3, pb3.reshape(1, -1))
    return logits[0]


# speedup vs baseline: 1.2260x; 1.0308x over previous
import functools

import jax
import jax.numpy as jnp
from jax import lax
from jax.experimental import pallas as pl
from jax.experimental.pallas import tpu as pltpu
from jax.experimental.pallas import tpu_sc as plsc

N_NODES = 10000
N_ACC = 10240
NUM_CORES = 2
NUM_SUBCORES = 16
K_EDGES = 128
ROWS_PER_TILE = N_ACC // NUM_SUBCORES
CPB = 4

_HIGH = jax.lax.Precision.HIGHEST


def _dot(a, b):
    return jax.lax.dot_general(a, b, (((1,), (0,)), ((), ())),
                               precision=_HIGH,
                               preferred_element_type=jnp.float32)


def _sc_segment_sum(table, gidx, sidx, zeros, d, n_chunks):
    mesh = plsc.VectorSubcoreMesh(core_axis_name="c", subcore_axis_name="s")
    n_blocks = n_chunks // CPB
    assert n_chunks % (2 * CPB) == 0

    @functools.partial(
        pl.kernel,
        out_type=jax.ShapeDtypeStruct((N_ACC, 128), jnp.float32),
        mesh=mesh,
        scratch_types=(
            [pltpu.VMEM((CPB, K_EDGES), jnp.int32) for _ in range(4)]
            + [pltpu.VMEM((K_EDGES, d), jnp.float32) for _ in range(CPB)]
            + [pltpu.VMEM_SHARED((N_ACC, d), jnp.float32),
               pltpu.VMEM_SHARED((N_ACC, d), jnp.float32)]
            + [pltpu.SemaphoreType.DMA for _ in range(CPB + 2)]
        ),
        compiler_params=pltpu.CompilerParams(use_tc_tiling_on_sc=False),
    )
    def seg_kernel(table_hbm, gidx_hbm, sidx_hbm, zeros_hbm, out_hbm, *rest):
        gbuf = rest[0:2]
        dbuf = rest[2:4]
        rows = rest[4:4 + CPB]
        acc_sh = rest[4 + CPB]
        tbl_sh = rest[5 + CPB]
        gsem = rest[6 + CPB:6 + 2 * CPB]
        isem = rest[6 + 2 * CPB:]
        c = lax.axis_index("c")
        s = lax.axis_index("s")
        rslc = pl.ds(s * ROWS_PER_TILE, ROWS_PER_TILE)
        cslc = pl.ds(c * d, d)
        pltpu.sync_copy(zeros_hbm.at[rslc], acc_sh.at[rslc])
        pltpu.sync_copy(table_hbm.at[rslc, cslc], tbl_sh.at[rslc])
        pltpu.sync_copy(gidx_hbm.at[s, pl.ds(0, CPB)], gbuf[0])
        pltpu.sync_copy(sidx_hbm.at[s, pl.ds(0, CPB)], dbuf[0])
        pltpu.make_async_copy(gidx_hbm.at[s, pl.ds(CPB, CPB)], gbuf[1],
                              isem[1]).start()
        pltpu.make_async_copy(sidx_hbm.at[s, pl.ds(CPB, CPB)], dbuf[1],
                              isem[1]).start()
        plsc.subcore_barrier()

        for j in range(CPB):
            pltpu.make_async_copy(tbl_sh.at[gbuf[0].at[j]], rows[j],
                                  gsem[j]).start()

        def half_step(blk_off, p):
            q = 1 - p
            pltpu.make_async_copy(gidx_hbm.at[s, pl.ds(0, CPB)], gbuf[q],
                                  isem[q]).wait()
            pltpu.make_async_copy(sidx_hbm.at[s, pl.ds(0, CPB)], dbuf[q],
                                  isem[q]).wait()
            for j in range(CPB):
                pltpu.make_async_copy(tbl_sh.at[gbuf[p].at[j]], rows[j],
                                      gsem[j]).wait()
                pltpu.sync_copy(rows[j], acc_sh.at[dbuf[p].at[j]], add=True)
                pltpu.make_async_copy(tbl_sh.at[gbuf[q].at[j]], rows[j],
                                      gsem[j]).start()
            nxt = pl.ds(lax.rem(blk_off + 2, n_blocks) * CPB, CPB)
            pltpu.make_async_copy(gidx_hbm.at[s, nxt], gbuf[p],
                                  isem[p]).start()
            pltpu.make_async_copy(sidx_hbm.at[s, nxt], dbuf[p],
                                  isem[p]).start()

        @pl.loop(0, n_blocks // 2)
        def _(t):
            half_step(2 * t, 0)
            half_step(2 * t + 1, 1)

        for j in range(CPB):
            pltpu.make_async_copy(tbl_sh.at[gbuf[0].at[j]], rows[j],
                                  gsem[j]).wait()
        pltpu.make_async_copy(gidx_hbm.at[s, pl.ds(0, CPB)], gbuf[1],
                              isem[1]).wait()
        pltpu.make_async_copy(sidx_hbm.at[s, pl.ds(0, CPB)], dbuf[1],
                              isem[1]).wait()

        plsc.subcore_barrier()
        pltpu.sync_copy(acc_sh.at[rslc], out_hbm.at[rslc, cslc])

    return seg_kernel(table, gidx, sidx, zeros)


def _sc_degree(sidx, zeros16, n_chunks):
    mesh = plsc.VectorSubcoreMesh(core_axis_name="c", subcore_axis_name="s")
    half = n_chunks // 2
    n_blocks = half // CPB

    @functools.partial(
        pl.kernel,
        out_type=jax.ShapeDtypeStruct((N_ACC, 128), jnp.float32),
        mesh=mesh,
        scratch_types=(
            [pltpu.VMEM((CPB, K_EDGES), jnp.int32) for _ in range(2)]
            + [pltpu.VMEM((K_EDGES, 16), jnp.float32),
               pltpu.VMEM_SHARED((N_ACC, 16), jnp.float32)]
            + [pltpu.SemaphoreType.DMA for _ in range(2)]
        ),
        compiler_params=pltpu.CompilerParams(use_tc_tiling_on_sc=False),
    )
    def deg_kernel(sidx_hbm, zeros_hbm, out_hbm, dbuf0, dbuf1, ones_v,
                   deg_sh, isem0, isem1):
        dbuf = (dbuf0, dbuf1)
        isem = (isem0, isem1)
        c = lax.axis_index("c")
        s = lax.axis_index("s")
        rslc = pl.ds(s * ROWS_PER_TILE, ROWS_PER_TILE)
        base = c * half
        pltpu.sync_copy(zeros_hbm.at[rslc], deg_sh.at[rslc])

        @pl.loop(0, K_EDGES)
        def _(i):
            ones_v[i] = jnp.full((16,), 1.0, jnp.float32)

        pltpu.sync_copy(sidx_hbm.at[s, pl.ds(base, CPB)], dbuf[0])
        pltpu.make_async_copy(sidx_hbm.at[s, pl.ds(base + CPB, CPB)], dbuf[1],
                              isem[1]).start()
        plsc.subcore_barrier()

        def half_step(blk_off, p):
            q = 1 - p
            pltpu.make_async_copy(sidx_hbm.at[s, pl.ds(0, CPB)], dbuf[q],
                                  isem[q]).wait()
            for j in range(CPB):
                pltpu.sync_copy(ones_v, deg_sh.at[dbuf[p].at[j]], add=True)
            nxt = pl.ds(base + lax.rem(blk_off + 2, n_blocks) * CPB, CPB)
            pltpu.make_async_copy(sidx_hbm.at[s, nxt], dbuf[p],
                                  isem[p]).start()

        @pl.loop(0, n_blocks // 2)
        def _(t):
            half_step(2 * t, 0)
            half_step(2 * t + 1, 1)

        pltpu.make_async_copy(sidx_hbm.at[s, pl.ds(0, CPB)], dbuf[1],
                              isem[1]).wait()
        plsc.subcore_barrier()
        pltpu.sync_copy(deg_sh.at[rslc], out_hbm.at[rslc, pl.ds(c * 16, 16)])

    return deg_kernel(sidx, zeros16)


ROW_BLK = 2000


def _tc_project(x, Ws, Wn):
    n, d_in = x.shape
    d_s = Ws.shape[1]
    d_n = Wn.shape[1]
    h2 = d_n // 2

    def body(x_ref, ws_ref, wn_ref, hs_ref, hw_ref):
        xb = x_ref[...]
        hs_ref[...] = _dot(xb, ws_ref[...])
        hw_ref[:, :d_n] = _dot(xb, wn_ref[...])

    return pl.pallas_call(
        body,
        grid=(n // ROW_BLK,),
        in_specs=[
            pl.BlockSpec((ROW_BLK, d_in), lambda i: (i, 0)),
            pl.BlockSpec((d_in, d_s), lambda i: (0, 0)),
            pl.BlockSpec((d_in, d_n), lambda i: (0, 0)),
        ],
        out_specs=[
            pl.BlockSpec((ROW_BLK, d_s), lambda i: (i, 0)),
            pl.BlockSpec((ROW_BLK, 128), lambda i: (i, 0)),
        ],
        out_shape=[
            jax.ShapeDtypeStruct((n, d_s), jnp.float32),
            jax.ShapeDtypeStruct((N_ACC, 128), jnp.float32),
        ],
    )(x, Ws, Wn)


def _tc_combine_project(hs, acc, deg, b, Ws_next, Wn_next):
    n, d = hs.shape
    d_s = Ws_next.shape[1]
    d_n = Wn_next.shape[1]
    h2 = d_n // 2

    def body(hs_ref, acc_ref, deg_ref, b_ref, ws_ref, wn_ref, hs2_ref,
             hw2_ref):
        degv = deg_ref[:, 0:1] + deg_ref[:, 16:17]
        inv = 1.0 / jnp.maximum(degv, 1.0)
        neigh = acc_ref[:, :d] * inv
        h = jnp.maximum(hs_ref[...] + neigh + b_ref[...], 0.0)
        hs2_ref[...] = _dot(h, ws_ref[...])
        hw2_ref[:, :d_n] = _dot(h, wn_ref[...])

    return pl.pallas_call(
        body,
        grid=(n // ROW_BLK,),
        in_specs=[
            pl.BlockSpec((ROW_BLK, d), lambda i: (i, 0)),
            pl.BlockSpec((ROW_BLK, 128), lambda i: (i, 0)),
            pl.BlockSpec((ROW_BLK, 128), lambda i: (i, 0)),
            pl.BlockSpec((1, d), lambda i: (0, 0)),
            pl.BlockSpec((d, d_s), lambda i: (0, 0)),
            pl.BlockSpec((d, d_n), lambda i: (0, 0)),
        ],
        out_specs=[
            pl.BlockSpec((ROW_BLK, d_s), lambda i: (i, 0)),
            pl.BlockSpec((ROW_BLK, 128), lambda i: (i, 0)),
        ],
        out_shape=[
            jax.ShapeDtypeStruct((n, d_s), jnp.float32),
            jax.ShapeDtypeStruct((N_ACC, 128), jnp.float32),
        ],
    )(hs, acc, deg, b, Ws_next, Wn_next)


def _tc_final(hs3, acc3, deg, b3, pW1, pb1, pW2, pb2, pW3, pb3):
    n, d = hs3.shape
    a_dim = pW3.shape[1]

    def body(hs_ref, acc_ref, deg_ref, b_ref, w1_ref, b1_ref, w2_ref, b2_ref,
             w3_ref, b3_ref, out_ref):
        degv = deg_ref[:n, 0:1] + deg_ref[:n, 16:17]
        inv = 1.0 / jnp.maximum(degv, 1.0)
        neigh = acc_ref[:n, :d] * inv
        h = jnp.maximum(hs_ref[...] + neigh + b_ref[...], 0.0)
        g = jnp.sum(h, axis=0, keepdims=True) * (1.0 / n)
        l1 = jnp.maximum(_dot(g, w1_ref[...]) + b1_ref[...], 0.0)
        l2 = jnp.maximum(_dot(l1, w2_ref[...]) + b2_ref[...], 0.0)
        out_ref[...] = _dot(l2, w3_ref[...]) + b3_ref[...]

    return pl.pallas_call(
        body,
        out_shape=jax.ShapeDtypeStruct((1, a_dim), jnp.float32),
    )(hs3, acc3, deg, b3, pW1, pb1, pW2, pb2, pW3, pb3)


def kernel(x, edge_index, Ws1, Wn1, b1, Ws2, Wn2, b2, Ws3, Wn3, b3,
           pW1, pb1, pW2, pb2, pW3, pb3):
    n = x.shape[0]
    e = edge_index.shape[1]
    src = edge_index[0]
    dst = edge_index[1]

    n_chunks = -(-e // (NUM_SUBCORES * K_EDGES))
    n_chunks = -(-n_chunks // (2 * CPB)) * (2 * CPB)
    e_pad = NUM_SUBCORES * K_EDGES * n_chunks
    pad = e_pad - e
    pad_rows = n + (jnp.arange(pad, dtype=jnp.int32) % (N_ACC - n))
    idx_c = (NUM_SUBCORES, n_chunks, K_EDGES)
    src_c = jnp.concatenate([src, jnp.zeros((pad,), jnp.int32)]).reshape(idx_c)
    dst_c = jnp.concatenate([dst, pad_rows]).reshape(idx_c)

    zeros64 = jnp.zeros((N_ACC, 64), jnp.float32)
    zeros32 = jnp.zeros((N_ACC, 32), jnp.float32)
    zeros16 = jnp.zeros((N_ACC, 16), jnp.float32)

    b1r = b1.reshape(1, -1)
    b2r = b2.reshape(1, -1)
    b3r = b3.reshape(1, -1)

    deg = _sc_degree(dst_c, zeros16, n_chunks)

    hs1, hw1p = _tc_project(x, Ws1, Wn1)
    acc1 = _sc_segment_sum(hw1p, src_c, dst_c, zeros64, 64, n_chunks)
    hs2, hw2p = _tc_combine_project(hs1, acc1, deg, b1r, Ws2, Wn2)
    acc2 = _sc_segment_sum(hw2p, src_c, dst_c, zeros64, 64, n_chunks)
    hs3, hw3p = _tc_combine_project(hs2, acc2, deg, b2r, Ws3, Wn3)
    acc3 = _sc_segment_sum(hw3p, src_c, dst_c, zeros32, 32, n_chunks)
    logits = _tc_final(hs3, acc3, deg, b3r,
                       pW1, pb1.reshape(1, -1), pW2, pb2.reshape(1, -1),
                       pW3, pb3.reshape(1, -1))
    return logits[0]
